# Initial kernel scaffold; baseline (speedup 1.0000x reference)
import jax
import jax.numpy as jnp
from jax.experimental import pallas as pl


def kernel(rpn_cls_score, gt_boxes, im_info, anchors):
    raise NotImplementedError("write your pallas kernel here")


# speedup vs baseline: 4.2115x; 4.2115x over previous
import functools

import jax
import jax.numpy as jnp
import numpy as np
from jax import lax
from jax.experimental import pallas as pl
from jax.experimental.pallas import tpu as pltpu
from jax.experimental.pallas import tpu_sc as plsc

RPN_NEG_OVERLAP = 0.3
RPN_POS_OVERLAP = 0.7
RPN_BATCHSIZE = 256

_NT = 16
_L = 16
_BIG = np.int32(2 ** 30)
_LN2 = 0.6931471805599453
_LOG2_COEF = (
    -3.2352098285508246, 7.08510274970646, -7.396151552267666,
    5.673521559409963, -2.914492700517339, 0.9507418392586514,
    -0.178109744191516, 0.01459848929291612,
)


def _np_threefry_uniform(seed: int, n: int) -> np.ndarray:
    def rotl(x, d):
        return ((x << np.uint32(d)) | (x >> np.uint32(32 - d))).astype(np.uint32)

    rotations = ((13, 15, 26, 6), (17, 29, 16, 24))
    k1, k2 = np.uint32(0), np.uint32(seed)
    ks = (k1, k2, k1 ^ k2 ^ np.uint32(0x1BD11BDA))
    x0 = (np.zeros(n, np.uint32) + ks[0]).astype(np.uint32)
    x1 = (np.arange(n, dtype=np.uint32) + ks[1]).astype(np.uint32)
    for i in range(5):
        for r in rotations[i % 2]:
            x0 = (x0 + x1).astype(np.uint32)
            x1 = rotl(x1, r)
            x1 = (x1 ^ x0).astype(np.uint32)
        x0 = (x0 + ks[(i + 1) % 3]).astype(np.uint32)
        x1 = (x1 + ks[(i + 2) % 3] + np.uint32(i + 1)).astype(np.uint32)
    bits = x0 ^ x1
    f = ((bits >> np.uint32(9)) | np.uint32(0x3F800000)).view(np.float32)
    return np.maximum(np.float32(0.0), f - np.float32(1.0))


@functools.lru_cache(maxsize=None)
def _selection_consts(A: int):
    perms, ranks = [], []
    for key in (42, 43):
        prio = _np_threefry_uniform(key, A)
        perm = np.argsort(-prio, kind="stable").astype(np.int32)
        rank = np.empty(A, np.int32)
        rank[perm] = np.arange(A, dtype=np.int32)
        perms.append(perm)
        ranks.append(rank)
    return np.concatenate(perms), np.concatenate(ranks)


def _ln(x):
    bits = plsc.bitcast(x, jnp.int32)
    e = ((bits >> 23) & 0xFF) - 127
    m = plsc.bitcast((bits & 0x007FFFFF) | 0x3F800000, jnp.float32)
    p = jnp.full((_L,), _LOG2_COEF[-1], jnp.float32)
    for c in _LOG2_COEF[-2::-1]:
        p = p * m + c
    return (e.astype(jnp.float32) + p) * _LN2


def _build_sc_call(A: int, G: int):
    N0 = ((A + _NT - 1) // _NT + 127) // 128 * 128
    NLAST = A - (_NT - 1) * N0
    assert 0 < NLAST <= N0 and NLAST % 16 == 0 and N0 % 128 == 0
    assert G % _L == 0
    NC0, NCL = N0 // _L, NLAST // _L
    f32, i32 = jnp.float32, jnp.int32

    def body(anc, gt, imv, perms, ranks,
             lab_o, tg_o, inw_o, outw_o, flg_o,
             ax_v, gt_v, im_v, qa_v, ovm_v, gmax_v, gtmp_v,
             maxov_v, argm_v, ins_v, flags_v, prm_v, srt_v, cum_v, rnk_v,
             lab_v, tg_v, inw_v, outw_v, t16_v, s16_v,
             sh_gmax, sh_scal, sh_scal2, sem):
        wid = lax.axis_index("s")
        o_t = wid * N0
        nch = jnp.where(wid == _NT - 1, NCL, NC0)
        lane = lax.iota(i32, _L)

        def dma_in(SZ):
            for c in range(4):
                pltpu.sync_copy(anc.at[pl.ds(c * A + o_t, SZ)],
                                ax_v.at[pl.ds(c * N0, SZ)])
            for r in range(2):
                pltpu.sync_copy(perms.at[pl.ds(r * A + o_t, SZ)],
                                prm_v.at[pl.ds(r * N0, SZ)])
                pltpu.sync_copy(ranks.at[pl.ds(r * A + o_t, SZ)],
                                rnk_v.at[pl.ds(r * N0, SZ)])

        pl.when(wid < _NT - 1)(lambda: dma_in(N0))
        pl.when(wid == _NT - 1)(lambda: dma_in(NLAST))
        pltpu.sync_copy(gt, gt_v)
        pltpu.sync_copy(imv, im_v)

        def qa_step(j, _):
            ds = pl.ds(j * _L, _L)
            qw = gt_v[pl.ds(2 * G + j * _L, _L)] - gt_v[pl.ds(0 * G + j * _L, _L)] + 1.0
            qh = gt_v[pl.ds(3 * G + j * _L, _L)] - gt_v[pl.ds(1 * G + j * _L, _L)] + 1.0
            qa_v[ds] = qw * qh
            return 0

        lax.fori_loop(0, G // _L, qa_step, 0)

        def gm_init(g, _):
            gmax_v[pl.ds(g * _L, _L)] = jnp.full((_L,), -1.0, f32)
            return 0

        lax.fori_loop(0, G, gm_init, 0)

        imw = im_v[pl.ds(0, _L)]
        imh = im_v[pl.ds(_L, _L)]

        def p1_chunk(c, _):
            b = c * _L
            ax1 = ax_v[pl.ds(0 * N0 + b, _L)]
            ay1 = ax_v[pl.ds(1 * N0 + b, _L)]
            ax2 = ax_v[pl.ds(2 * N0 + b, _L)]
            ay2 = ax_v[pl.ds(3 * N0 + b, _L)]
            ins = (ax1 >= 0.0) & (ay1 >= 0.0) & (ax2 < imw) & (ay2 < imh)
            ins_v[pl.ds(b, _L)] = ins.astype(i32)
            ba = (ax2 - ax1 + 1.0) * (ay2 - ay1 + 1.0)

            def g_step(j, car):
                cmax, carg = car
                gb = j * _L
                gx1v = gt_v[pl.ds(0 * G + gb, _L)]
                gy1v = gt_v[pl.ds(1 * G + gb, _L)]
                gx2v = gt_v[pl.ds(2 * G + gb, _L)]
                gy2v = gt_v[pl.ds(3 * G + gb, _L)]
                qav = qa_v[pl.ds(gb, _L)]
                for k in range(_L):
                    g = gb + k
                    iw = jnp.minimum(ax2, gx2v[k]) - jnp.maximum(ax1, gx1v[k]) + 1.0
                    ih = jnp.minimum(ay2, gy2v[k]) - jnp.maximum(ay1, gy1v[k]) + 1.0
                    inter = jnp.maximum(iw, 0.0) * jnp.maximum(ih, 0.0)
                    ov = inter / (ba + qav[k] - inter)
                    upd = ov > cmax
                    cmax = jnp.where(upd, ov, cmax)
                    carg = jnp.where(upd, g, carg)
                    ovm = jnp.where(ins, ov, -1.0)
                    ovm_v[pl.ds(g * N0 + b, _L)] = ovm
                    gds = pl.ds(g * _L, _L)
                    gmax_v[gds] = jnp.maximum(gmax_v[gds], ovm)
                return cmax, carg

            cmax, carg = lax.fori_loop(
                0, G // _L, g_step,
                (jnp.full((_L,), -1.0, f32), jnp.zeros((_L,), i32)))
            maxov_v[pl.ds(b, _L)] = cmax
            argm_v[pl.ds(b, _L)] = carg
            return 0

        lax.fori_loop(0, nch, p1_chunk, 0)

        pltpu.sync_copy(gmax_v, sh_gmax.at[pl.ds(wid * (G * _L), G * _L)])
        plsc.subcore_barrier()

        def gm_red(t, _):
            pltpu.sync_copy(sh_gmax.at[pl.ds(t * (G * _L), G * _L)], gtmp_v)

            def row(g, _):
                gds = pl.ds(g * _L, _L)
                gmax_v[gds] = jnp.maximum(gmax_v[gds], gtmp_v[gds])
                return 0

            lax.fori_loop(0, G, row, 0)
            return 0

        lax.fori_loop(0, _NT, gm_red, 0)

        def gm_splat(g, _):
            gds = pl.ds(g * _L, _L)
            gmax_v[gds] = jnp.full((_L,), jnp.max(gmax_v[gds]), f32)
            return 0

        lax.fori_loop(0, G, gm_splat, 0)

        def p2_chunk(c, _):
            b = c * _L
            ins = ins_v[pl.ds(b, _L)] == 1
            mov = maxov_v[pl.ds(b, _L)]

            def g_step(j, acc):
                for k in range(4):
                    g = j * 4 + k
                    eq = ovm_v[pl.ds(g * N0 + b, _L)] == gmax_v[pl.ds(g * _L, _L)]
                    acc = acc | eq.astype(i32)
                return acc

            isga = lax.fori_loop(0, G // 4, g_step, jnp.zeros((_L,), i32))
            isga_b = (isga == 1) & ins
            fg = isga_b | (ins & (mov >= RPN_POS_OVERLAP))
            bg = ins & (mov < RPN_NEG_OVERLAP) & (~isga_b)
            flags_v[pl.ds(b, _L)] = fg.astype(i32) | (bg.astype(i32) << 1)
            return 0

        lax.fori_loop(0, nch, p2_chunk, 0)

        def flg_out(SZ):
            pltpu.sync_copy(flags_v.at[pl.ds(0, SZ)], flg_o.at[pl.ds(o_t, SZ)])

        pl.when(wid < _NT - 1)(lambda: flg_out(N0))
        pl.when(wid == _NT - 1)(lambda: flg_out(NLAST))
        plsc.subcore_barrier()

        def gathers(SZ):
            hs = []
            for r in range(2):
                off = 0
                while off < SZ:
                    n = min(128, SZ - off)
                    hs.append(pltpu.async_copy(
                        flg_o.at[prm_v.at[pl.ds(r * N0 + off, n)]],
                        srt_v.at[pl.ds(r * N0 + off, n)], sem))
                    off += n
            for h in hs:
                h.wait()

        pl.when(wid < _NT - 1)(lambda: gathers(N0))
        pl.when(wid == _NT - 1)(lambda: gathers(NLAST))

        def csum(r):
            def cstep(c, carry):
                ds = pl.ds(r * N0 + c * _L, _L)
                bits = (srt_v[ds] >> r) & 1
                cs = plsc.cumsum(bits) + carry
                cum_v[ds] = cs
                return jnp.max(cs)

            return lax.fori_loop(0, nch, cstep, jnp.int32(0))

        totfg = csum(0)
        totbg = csum(1)
        t16_v[...] = jnp.where(lane == 0, totfg, jnp.where(lane == 1, totbg, 0))
        pltpu.sync_copy(t16_v, sh_scal.at[pl.ds(wid * _L, _L)])
        plsc.subcore_barrier()

        pltpu.sync_copy(sh_scal, s16_v)
        fgt = plsc.load_gather(s16_v, [lane * _L])
        bgt = plsc.load_gather(s16_v, [lane * _L + 1])
        count_fg = jnp.sum(fgt)
        count_bg = jnp.sum(bgt)
        off_fg = jnp.sum(jnp.where(lane < wid, fgt, 0))
        off_bg = jnp.sum(jnp.where(lane < wid, bgt, 0))
        n_fg = jnp.minimum(count_fg, RPN_BATCHSIZE // 2)
        num_bg = RPN_BATCHSIZE - n_fg
        n_bg = jnp.minimum(count_bg, num_bg)
        posv = 1.0 / jnp.full((_L,), n_fg + n_bg, i32).astype(f32)

        def findj(r, tgt):
            def cstep(c, cand):
                ds = pl.ds(r * N0 + c * _L, _L)
                bits = (srt_v[ds] >> r) & 1
                m = (cum_v[ds] == tgt) & (bits == 1)
                pos = jnp.where(m, c * _L + lane, _BIG)
                return jnp.minimum(cand, jnp.min(pos))

            j = lax.fori_loop(0, nch, cstep, jnp.int32(_BIG))
            return jnp.where(j == _BIG, _BIG, o_t + j)

        rcf = findj(0, RPN_BATCHSIZE // 2 - off_fg)
        rcb = findj(1, num_bg - off_bg)
        t16_v[...] = jnp.where(lane == 0, rcf, jnp.where(lane == 1, rcb, _BIG))
        pltpu.sync_copy(t16_v, sh_scal2.at[pl.ds(wid * _L, _L)])
        plsc.subcore_barrier()

        pltpu.sync_copy(sh_scal2, s16_v)
        rfg = jnp.min(plsc.load_gather(s16_v, [lane * _L]))
        rbg = jnp.min(plsc.load_gather(s16_v, [lane * _L + 1]))

        def fin_chunk(c, _):
            b = c * _L
            ds = pl.ds(b, _L)
            f = flags_v[ds]
            kfg = ((f & 1) == 1) & (rnk_v[pl.ds(0 * N0 + b, _L)] <= rfg)
            kbg = (((f >> 1) & 1) == 1) & (rnk_v[pl.ds(1 * N0 + b, _L)] <= rbg)
            lab_v[ds] = jnp.where(kfg, 1, jnp.where(kbg, 0, -1)).astype(i32)
            am = argm_v[ds]
            gx1 = plsc.load_gather(gt_v, [am + (0 * G)])
            gy1 = plsc.load_gather(gt_v, [am + (1 * G)])
            gx2 = plsc.load_gather(gt_v, [am + (2 * G)])
            gy2 = plsc.load_gather(gt_v, [am + (3 * G)])
            ax1 = ax_v[pl.ds(0 * N0 + b, _L)]
            ay1 = ax_v[pl.ds(1 * N0 + b, _L)]
            ax2 = ax_v[pl.ds(2 * N0 + b, _L)]
            ay2 = ax_v[pl.ds(3 * N0 + b, _L)]
            ew = ax2 - ax1 + 1.0
            eh = ay2 - ay1 + 1.0
            ecx = ax1 + 0.5 * ew
            ecy = ay1 + 0.5 * eh
            gw = gx2 - gx1 + 1.0
            gh = gy2 - gy1 + 1.0
            gcx = gx1 + 0.5 * gw
            gcy = gy1 + 0.5 * gh
            ins = ins_v[ds] == 1
            zf = jnp.zeros((_L,), f32)
            dx = jnp.where(ins, (gcx - ecx) / ew, zf)
            dy = jnp.where(ins, (gcy - ecy) / eh, zf)
            dw = jnp.where(ins, _ln(gw / ew), zf)
            dh = jnp.where(ins, _ln(gh / eh), zf)
            rows4 = (b + lane) * 4
            ivw = jnp.where(kfg, 1.0, 0.0).astype(f32)
            ovw = jnp.where(kfg | kbg, posv, zf)
            for comp, val in enumerate((dx, dy, dw, dh)):
                plsc.store_scatter(tg_v, [rows4 + comp], val)
                plsc.store_scatter(inw_v, [rows4 + comp], ivw)
                plsc.store_scatter(outw_v, [rows4 + comp], ovw)
            return 0

        lax.fori_loop(0, nch, fin_chunk, 0)

        def dma_out(SZ):
            pltpu.sync_copy(lab_v.at[pl.ds(0, SZ)], lab_o.at[pl.ds(o_t, SZ)])
            pltpu.sync_copy(tg_v.at[pl.ds(0, 4 * SZ)], tg_o.at[pl.ds(4 * o_t, 4 * SZ)])
            pltpu.sync_copy(inw_v.at[pl.ds(0, 4 * SZ)], inw_o.at[pl.ds(4 * o_t, 4 * SZ)])
            pltpu.sync_copy(outw_v.at[pl.ds(0, 4 * SZ)], outw_o.at[pl.ds(4 * o_t, 4 * SZ)])

        pl.when(wid < _NT - 1)(lambda: dma_out(N0))
        pl.when(wid == _NT - 1)(lambda: dma_out(NLAST))

    mesh = plsc.VectorSubcoreMesh(
        core_axis_name="c", subcore_axis_name="s", num_cores=1,
        num_subcores=_NT)
    return pl.kernel(
        body,
        out_type=[
            jax.ShapeDtypeStruct((A,), jnp.int32),
            jax.ShapeDtypeStruct((4 * A,), jnp.float32),
            jax.ShapeDtypeStruct((4 * A,), jnp.float32),
            jax.ShapeDtypeStruct((4 * A,), jnp.float32),
            jax.ShapeDtypeStruct((A,), jnp.int32),
        ],
        mesh=mesh,
        scratch_types=[
            pltpu.VMEM((4 * N0,), f32),
            pltpu.VMEM((4 * G,), f32),
            pltpu.VMEM((2 * _L,), f32),
            pltpu.VMEM((G,), f32),
            pltpu.VMEM((G * N0,), f32),
            pltpu.VMEM((G * _L,), f32),
            pltpu.VMEM((G * _L,), f32),
            pltpu.VMEM((N0,), f32),
            pltpu.VMEM((N0,), i32),
            pltpu.VMEM((N0,), i32),
            pltpu.VMEM((N0,), i32),
            pltpu.VMEM((2 * N0,), i32),
            pltpu.VMEM((2 * N0,), i32),
            pltpu.VMEM((2 * N0,), i32),
            pltpu.VMEM((2 * N0,), i32),
            pltpu.VMEM((N0,), i32),
            pltpu.VMEM((4 * N0,), f32),
            pltpu.VMEM((4 * N0,), f32),
            pltpu.VMEM((4 * N0,), f32),
            pltpu.VMEM((_L,), i32),
            pltpu.VMEM((_NT * _L,), i32),
            pltpu.VMEM_SHARED((_NT * G * _L,), f32),
            pltpu.VMEM_SHARED((_NT * _L,), i32),
            pltpu.VMEM_SHARED((_NT * _L,), i32),
            pltpu.SemaphoreType.DMA,
        ],
        compiler_params=pltpu.CompilerParams(needs_layout_passes=False),
        name="anchor_target_sc",
    )


def kernel(rpn_cls_score, gt_boxes, im_info, anchors):
    A = anchors.shape[0]
    G = gt_boxes.shape[0]
    H, W = rpn_cls_score.shape[1], rpn_cls_score.shape[2]
    na = A // (H * W)
    perms, ranks = _selection_consts(A)
    anc = anchors.T.reshape(-1)
    gt = gt_boxes[:, :4].T.reshape(-1)
    imv = jnp.concatenate(
        [jnp.full((_L,), im_info[1]), jnp.full((_L,), im_info[0])])
    lab, tg, inw, outw, _ = _build_sc_call(A, G)(
        anc, gt, imv, jnp.asarray(perms), jnp.asarray(ranks))
    rpn_labels = lab.reshape(1, H, W, na).transpose(0, 3, 1, 2).reshape(
        1, 1, na * H, W)
    return (
        rpn_labels,
        tg.reshape(1, H, W, na * 4),
        inw.reshape(1, H, W, na * 4),
        outw.reshape(1, H, W, na * 4),
    )
